# baseline (speedup 1.0000x reference)
import functools

import jax
import jax.numpy as jnp
from jax.experimental import pallas as pl
from jax.experimental.pallas import tpu as pltpu

N = 10000
E = 160000
T = 4
D = 128
H1 = 32
HEADS = 4
GH = 64


def _gru_head_body(e0, e1, e2, e3, wih0, whh0, bih0, bhh0,
                   wih1, whh1, bih1, bhh1, wd1, bd1, wd2, bd2, out_ref):
    blk = e0.shape[0]
    embs = (e0[...], e1[...], e2[...], e3[...])
    h0 = jnp.zeros((blk, GH), jnp.float32)
    h1 = jnp.zeros((blk, GH), jnp.float32)
    w_ih0 = wih0[...]
    w_hh0 = whh0[...]
    w_ih1 = wih1[...]
    w_hh1 = whh1[...]
    for t in range(T):
        gi = jnp.dot(embs[t], w_ih0, preferred_element_type=jnp.float32) + bih0[...]
        gh = jnp.dot(h0, w_hh0, preferred_element_type=jnp.float32) + bhh0[...]
        r = jax.nn.sigmoid(gi[:, :GH] + gh[:, :GH])
        z = jax.nn.sigmoid(gi[:, GH:2 * GH] + gh[:, GH:2 * GH])
        n = jnp.tanh(gi[:, 2 * GH:] + r * gh[:, 2 * GH:])
        h0 = (1.0 - z) * n + z * h0
        gi = jnp.dot(h0, w_ih1, preferred_element_type=jnp.float32) + bih1[...]
        gh = jnp.dot(h1, w_hh1, preferred_element_type=jnp.float32) + bhh1[...]
        r = jax.nn.sigmoid(gi[:, :GH] + gh[:, :GH])
        z = jax.nn.sigmoid(gi[:, GH:2 * GH] + gh[:, GH:2 * GH])
        n = jnp.tanh(gi[:, 2 * GH:] + r * gh[:, 2 * GH:])
        h1 = (1.0 - z) * n + z * h1
    hid = jax.nn.relu(jnp.dot(h1, wd1[...], preferred_element_type=jnp.float32)
                      + bd1[...])
    pred = jnp.dot(hid, wd2[...], preferred_element_type=jnp.float32) + bd2[...]
    out_ref[...] = pred


def _gru_head(embs, Wih0, Whh0, bih0, bhh0, Wih1, Whh1, bih1, bhh1,
              Wd1, bd1, Wd2, bd2):
    BLK = 400
    grid = (E // BLK,)
    eb = pl.BlockSpec((BLK, 2 * H1), lambda i: (i, 0))
    full = lambda shape: pl.BlockSpec(shape, lambda i: (0,) * len(shape))
    w_specs = [
        full((2 * H1, 3 * GH)), full((GH, 3 * GH)), full((1, 3 * GH)),
        full((1, 3 * GH)),
        full((GH, 3 * GH)), full((GH, 3 * GH)), full((1, 3 * GH)),
        full((1, 3 * GH)),
        full((GH, GH // 2)), full((1, GH // 2)), full((GH // 2, 1)),
        full((1, 1)),
    ]
    return pl.pallas_call(
        _gru_head_body,
        grid=grid,
        in_specs=[eb, eb, eb, eb] + w_specs,
        out_specs=pl.BlockSpec((BLK, 1), lambda i: (i, 0)),
        out_shape=jax.ShapeDtypeStruct((E, 1), jnp.float32),
    )(
        embs[0], embs[1], embs[2], embs[3],
        Wih0.T, Whh0.T, bih0[None], bhh0[None],
        Wih1.T, Whh1.T, bih1[None], bhh1[None],
        Wd1.T, bd1[None], Wd2.T, bd2[None],
    )


def _gat(x, src, dst, W, a_src, a_dst, b, heads, out_ch, concat):
    n = x.shape[0]
    h = (x @ W).reshape(n, heads, out_ch)
    e_src = jnp.sum(h * a_src[None], axis=-1)
    e_dst = jnp.sum(h * a_dst[None], axis=-1)
    m = jnp.max(e_src, axis=0) + jnp.max(e_dst, axis=0)
    m = jnp.maximum(m, 0.0)
    e = e_src[src] + e_dst[dst]
    e = jnp.where(e > 0, e, 0.2 * e)
    ex = jnp.exp(e - m[None])
    s = jax.ops.segment_sum(ex, dst, num_segments=n)
    num = jax.ops.segment_sum(h[src] * ex[..., None], dst, num_segments=n)
    out = num / (s[..., None] + 1e-16)
    out = out.reshape(n, heads * out_ch) if concat else out.mean(axis=1)
    return out + b


def _encoder(x, src, dst, W1, as1, ad1, b1, W2, as2, ad2, b2):
    h = _gat(x, src, dst, W1, as1, ad1, b1, HEADS, H1, True)
    h = jax.nn.elu(h)
    h = _gat(h, src, dst, W2, as2, ad2, b2, 1, H1, False)
    return h[src], h[dst]


def kernel(x_seq, common_edge_index, W1, as1, ad1, b1, W2, as2, ad2, b2,
           Wih0, Whh0, bih0, bhh0, Wih1, Whh1, bih1, bhh1, Wd1, bd1, Wd2,
           bd2):
    src = common_edge_index[0]
    dst = common_edge_index[1]
    embs = []
    for t in range(T):
        u, v = _encoder(x_seq[t], src, dst, W1, as1, ad1, b1, W2, as2, ad2,
                        b2)
        embs.append(jnp.concatenate([u, v], axis=-1))
    return _gru_head(embs, Wih0, Whh0, bih0, bhh0, Wih1, Whh1, bih1, bhh1,
                     Wd1, bd1, Wd2, bd2)


# speedup vs baseline: 14.4052x; 14.4052x over previous
import functools

import jax
import jax.numpy as jnp
from jax import lax
from jax.experimental import pallas as pl
from jax.experimental.pallas import tpu as pltpu
from jax.experimental.pallas import tpu_sc as plsc

N = 10000
E = 160000
T = 4
D = 128
H1 = 32
HEADS = 4
GH = 64

NP = 10112
BN = 2528
W1T = 144
W2T = 48
EK = 128
NCH = E // EK
NTILES = 32
STRIPE = NP // 16


def _k1_body(x, w1, asb, adb, tab, edp, mx):
    i = pl.program_id(0)
    h = jnp.dot(x[...], w1[...], preferred_element_type=jnp.float32)
    es = jnp.dot(h, asb[...], preferred_element_type=jnp.float32)
    ed = jnp.dot(h, adb[...], preferred_element_type=jnp.float32)
    nh = es.shape[1]
    tab[...] = jnp.concatenate(
        [h, es, jnp.zeros((h.shape[0], W1T - D - nh), jnp.float32)], axis=1)
    edp[...] = jnp.concatenate(
        [ed, jnp.zeros((h.shape[0], 16 - nh), jnp.float32)], axis=1)
    bm = jnp.concatenate([jnp.max(es, axis=0), jnp.max(ed, axis=0),
                          jnp.zeros((16 - 2 * nh,), jnp.float32)])[None]

    @pl.when(i == 0)
    def _():
        mx[...] = bm

    @pl.when(i != 0)
    def _():
        mx[...] = jnp.maximum(mx[...], bm)


def _dense1(xp, W1, As, Ad):
    return pl.pallas_call(
        _k1_body,
        grid=(NP // BN,),
        in_specs=[
            pl.BlockSpec((BN, D), lambda i: (i, 0)),
            pl.BlockSpec((D, D), lambda i: (0, 0)),
            pl.BlockSpec((D, HEADS), lambda i: (0, 0)),
            pl.BlockSpec((D, HEADS), lambda i: (0, 0)),
        ],
        out_specs=[
            pl.BlockSpec((BN, W1T), lambda i: (i, 0)),
            pl.BlockSpec((BN, 16), lambda i: (i, 0)),
            pl.BlockSpec((1, 16), lambda i: (0, 0)),
        ],
        out_shape=[
            jax.ShapeDtypeStruct((NP, W1T), jnp.float32),
            jax.ShapeDtypeStruct((NP, 16), jnp.float32),
            jax.ShapeDtypeStruct((1, 16), jnp.float32),
        ],
    )(xp, W1, As, Ad)


def _k4_body(p, b1, w2, asb, adb, tab, edp, mx):
    i = pl.program_id(0)
    agg = p[0] + p[1]
    num = agg[:, :D].reshape(-1, HEADS, H1)
    den = agg[:, D:D + HEADS] + 1e-16
    o = (num / den[..., None]).reshape(-1, D) + b1[...]
    o = jnp.where(o > 0, o, jnp.exp(o) - 1.0)
    hh = jnp.dot(o, w2[...], preferred_element_type=jnp.float32)
    es = jnp.dot(hh, asb[...], preferred_element_type=jnp.float32)
    ed = jnp.dot(hh, adb[...], preferred_element_type=jnp.float32)
    tab[...] = jnp.concatenate(
        [hh, es, jnp.zeros((hh.shape[0], W2T - H1 - 1), jnp.float32)], axis=1)
    edp[...] = jnp.concatenate(
        [ed, jnp.zeros((hh.shape[0], 15), jnp.float32)], axis=1)
    bm = jnp.concatenate([jnp.max(es, axis=0), jnp.max(ed, axis=0),
                          jnp.zeros((14,), jnp.float32)])[None]

    @pl.when(i == 0)
    def _():
        mx[...] = bm

    @pl.when(i != 0)
    def _():
        mx[...] = jnp.maximum(mx[...], bm)


def _combine1_dense2(parts, b1, W2, As2, Ad2):
    return pl.pallas_call(
        _k4_body,
        grid=(NP // BN,),
        in_specs=[
            pl.BlockSpec((2, BN, W1T), lambda i: (0, i, 0)),
            pl.BlockSpec((1, D), lambda i: (0, 0)),
            pl.BlockSpec((D, H1), lambda i: (0, 0)),
            pl.BlockSpec((H1, 1), lambda i: (0, 0)),
            pl.BlockSpec((H1, 1), lambda i: (0, 0)),
        ],
        out_specs=[
            pl.BlockSpec((BN, W2T), lambda i: (i, 0)),
            pl.BlockSpec((BN, 16), lambda i: (i, 0)),
            pl.BlockSpec((1, 16), lambda i: (0, 0)),
        ],
        out_shape=[
            jax.ShapeDtypeStruct((NP, W2T), jnp.float32),
            jax.ShapeDtypeStruct((NP, 16), jnp.float32),
            jax.ShapeDtypeStruct((1, 16), jnp.float32),
        ],
    )(parts, b1, W2, As2, Ad2)


def _k6_body(p, b2, out):
    agg = p[0] + p[1]
    num = agg[:, :H1]
    den = agg[:, H1:H1 + 1] + 1e-16
    out[...] = num / den + b2[...]


def _combine2(parts, b2):
    return pl.pallas_call(
        _k6_body,
        grid=(NP // BN,),
        in_specs=[
            pl.BlockSpec((2, BN, W2T), lambda i: (0, i, 0)),
            pl.BlockSpec((1, H1), lambda i: (0, 0)),
        ],
        out_specs=pl.BlockSpec((BN, H1), lambda i: (i, 0)),
        out_shape=jax.ShapeDtypeStruct((NP, H1), jnp.float32),
    )(parts, b2)


def _make_gat_scatter(width, heads):
    ncolv = D if width == W1T else H1
    hc = ncolv // heads
    escol = ncolv
    mesh = plsc.VectorSubcoreMesh(core_axis_name="c", subcore_axis_name="s", num_cores=2, num_subcores=16)

    @functools.partial(
        pl.kernel,
        out_type=(jax.ShapeDtypeStruct((NP, width), jnp.float32),
                  jax.ShapeDtypeStruct((NP, width), jnp.float32)),
        mesh=mesh,
        compiler_params=pltpu.CompilerParams(use_tc_tiling_on_sc=False, needs_layout_passes=False),
        scratch_types=[
            pltpu.VMEM((EK,), jnp.int32),
            pltpu.VMEM((EK,), jnp.int32),
            pltpu.VMEM((EK, width), jnp.float32),
            pltpu.VMEM((EK, 16), jnp.float32),
            pltpu.VMEM((16, 16), jnp.float32),
            pltpu.VMEM_SHARED((NP, width), jnp.float32),
        ],
    )
    def gat_scatter(src_hbm, dst_hbm, tab_hbm, edp_hbm, mx_hbm, zer_hbm,
                    out0_hbm, out1_hbm, sidx, didx, rows, edr, mxv, acc):
        c = lax.axis_index("c")
        s = lax.axis_index("s")
        wid = s * 2 + c
        lane = lax.iota(jnp.int32, 16)

        pltpu.sync_copy(zer_hbm, acc.at[pl.ds(s * STRIPE, STRIPE)])
        pltpu.sync_copy(mx_hbm, mxv)
        mvec = [mxv[h] for h in range(heads)]
        plsc.subcore_barrier()

        nloop = (NCH + NTILES - 1) // NTILES

        def chunk(j, carry):
            cid = wid + NTILES * j

            @pl.when(cid < NCH)
            def _():
                base = cid * EK
                pltpu.sync_copy(src_hbm.at[pl.ds(base, EK)], sidx)
                pltpu.sync_copy(dst_hbm.at[pl.ds(base, EK)], didx)
                pltpu.sync_copy(tab_hbm.at[sidx], rows)
                pltpu.sync_copy(edp_hbm.at[didx], edr)
                for g in range(EK // 16):
                    eids = g * 16 + lane
                    for h in range(heads):
                        col = jnp.full((16,), escol + h, jnp.int32)
                        es = plsc.load_gather(rows, [eids, col])
                        ed = plsc.load_gather(
                            edr, [eids, jnp.full((16,), h, jnp.int32)])
                        e = es + ed
                        e = jnp.where(e > 0, e, 0.2 * e)
                        ex = jnp.exp(e - mvec[h])
                        plsc.store_scatter(rows, [eids, col], ex)

                def scale_col(cc, carry2):
                    mcol = escol + cc // hc
                    for g in range(EK // 16):
                        eids = g * 16 + lane
                        mult = plsc.load_gather(
                            rows, [eids, jnp.full((16,), mcol, jnp.int32)])
                        ccol = jnp.full((16,), cc, jnp.int32)
                        v = plsc.load_gather(rows, [eids, ccol])
                        plsc.store_scatter(rows, [eids, ccol], v * mult)
                    return carry2

                lax.fori_loop(0, ncolv, scale_col, 0)
                pltpu.sync_copy(rows, acc.at[didx], add=True)

            return carry

        lax.fori_loop(0, nloop, chunk, 0)
        plsc.subcore_barrier()

        @pl.when(c == 0)
        def _():
            pltpu.sync_copy(acc.at[pl.ds(s * STRIPE, STRIPE)],
                            out0_hbm.at[pl.ds(s * STRIPE, STRIPE)])

        @pl.when(c == 1)
        def _():
            pltpu.sync_copy(acc.at[pl.ds(s * STRIPE, STRIPE)],
                            out1_hbm.at[pl.ds(s * STRIPE, STRIPE)])

    return gat_scatter


_gat_scatter1 = _make_gat_scatter(W1T, HEADS)
_gat_scatter2 = _make_gat_scatter(W2T, 1)


def _make_edge_gather():
    mesh = plsc.VectorSubcoreMesh(core_axis_name="c", subcore_axis_name="s", num_cores=2, num_subcores=16)

    @functools.partial(
        pl.kernel,
        out_type=(jax.ShapeDtypeStruct((E, H1), jnp.float32),
                  jax.ShapeDtypeStruct((E, H1), jnp.float32)),
        mesh=mesh,
        compiler_params=pltpu.CompilerParams(use_tc_tiling_on_sc=False, needs_layout_passes=False),
        scratch_types=[
            pltpu.VMEM((EK,), jnp.int32),
            pltpu.VMEM((EK,), jnp.int32),
            pltpu.VMEM((EK, H1), jnp.float32),
            pltpu.VMEM((EK, H1), jnp.float32),
        ],
    )
    def edge_gather(src_hbm, dst_hbm, tab_hbm, u_hbm, v_hbm,
                    sidx, didx, urows, vrows):
        c = lax.axis_index("c")
        s = lax.axis_index("s")
        wid = s * 2 + c
        nloop = (NCH + NTILES - 1) // NTILES

        def chunk(j, carry):
            cid = wid + NTILES * j

            @pl.when(cid < NCH)
            def _():
                base = cid * EK
                pltpu.sync_copy(src_hbm.at[pl.ds(base, EK)], sidx)
                pltpu.sync_copy(dst_hbm.at[pl.ds(base, EK)], didx)
                pltpu.sync_copy(tab_hbm.at[sidx], urows)
                pltpu.sync_copy(tab_hbm.at[didx], vrows)
                pltpu.sync_copy(urows, u_hbm.at[pl.ds(base, EK)])
                pltpu.sync_copy(vrows, v_hbm.at[pl.ds(base, EK)])

            return carry

        lax.fori_loop(0, nloop, chunk, 0)

    return edge_gather


_edge_gather = _make_edge_gather()


def _gru_head_body(u0, u1, u2, u3, v0, v1, v2, v3, wihu0, wihv0, whh0,
                   bih0, bhh0, wih1, whh1, bih1, bhh1, wd1, bd1, wd2, bd2,
                   out_ref):
    blk = u0.shape[0]
    us = (u0[...], u1[...], u2[...], u3[...])
    vs = (v0[...], v1[...], v2[...], v3[...])
    h0 = jnp.zeros((blk, GH), jnp.float32)
    h1 = jnp.zeros((blk, GH), jnp.float32)
    w_ihu0 = wihu0[...]
    w_ihv0 = wihv0[...]
    w_hh0 = whh0[...]
    w_ih1 = wih1[...]
    w_hh1 = whh1[...]
    for t in range(T):
        gi = (jnp.dot(us[t], w_ihu0, preferred_element_type=jnp.float32)
              + jnp.dot(vs[t], w_ihv0, preferred_element_type=jnp.float32)
              + bih0[...])
        gh = jnp.dot(h0, w_hh0, preferred_element_type=jnp.float32) + bhh0[...]
        r = jax.nn.sigmoid(gi[:, :GH] + gh[:, :GH])
        z = jax.nn.sigmoid(gi[:, GH:2 * GH] + gh[:, GH:2 * GH])
        n = jnp.tanh(gi[:, 2 * GH:] + r * gh[:, 2 * GH:])
        h0 = (1.0 - z) * n + z * h0
        gi = jnp.dot(h0, w_ih1, preferred_element_type=jnp.float32) + bih1[...]
        gh = jnp.dot(h1, w_hh1, preferred_element_type=jnp.float32) + bhh1[...]
        r = jax.nn.sigmoid(gi[:, :GH] + gh[:, :GH])
        z = jax.nn.sigmoid(gi[:, GH:2 * GH] + gh[:, GH:2 * GH])
        n = jnp.tanh(gi[:, 2 * GH:] + r * gh[:, 2 * GH:])
        h1 = (1.0 - z) * n + z * h1
    hid = jax.nn.relu(jnp.dot(h1, wd1[...], preferred_element_type=jnp.float32)
                      + bd1[...])
    pred = jnp.dot(hid, wd2[...], preferred_element_type=jnp.float32) + bd2[...]
    out_ref[...] = pred


def _gru_head(us, vs, Wih0, Whh0, bih0, bhh0, Wih1, Whh1, bih1, bhh1,
              Wd1, bd1, Wd2, bd2):
    BLK = 400
    grid = (E // BLK,)
    eb = pl.BlockSpec((BLK, H1), lambda i: (i, 0))
    full = lambda shape: pl.BlockSpec(shape, lambda i: (0,) * len(shape))
    w_specs = [
        full((H1, 3 * GH)), full((H1, 3 * GH)), full((GH, 3 * GH)),
        full((1, 3 * GH)), full((1, 3 * GH)),
        full((GH, 3 * GH)), full((GH, 3 * GH)), full((1, 3 * GH)),
        full((1, 3 * GH)),
        full((GH, GH // 2)), full((1, GH // 2)), full((GH // 2, 1)),
        full((1, 1)),
    ]
    wih0t = Wih0.T
    return pl.pallas_call(
        _gru_head_body,
        grid=grid,
        in_specs=[eb] * 8 + w_specs,
        out_specs=pl.BlockSpec((BLK, 1), lambda i: (i, 0)),
        out_shape=jax.ShapeDtypeStruct((E, 1), jnp.float32),
    )(
        us[0], us[1], us[2], us[3], vs[0], vs[1], vs[2], vs[3],
        wih0t[:H1], wih0t[H1:], Whh0.T, bih0[None], bhh0[None],
        Wih1.T, Whh1.T, bih1[None], bhh1[None],
        Wd1.T, bd1[None], Wd2.T, bd2[None],
    )


def kernel(x_seq, common_edge_index, W1, as1, ad1, b1, W2, as2, ad2, b2,
           Wih0, Whh0, bih0, bhh0, Wih1, Whh1, bih1, bhh1, Wd1, bd1, Wd2,
           bd2):
    src = common_edge_index[0]
    dst = common_edge_index[1]
    xp = jnp.pad(x_seq, ((0, 0), (0, NP - N), (0, 0)))
    eye = jnp.repeat(jnp.eye(HEADS, dtype=jnp.float32), H1, axis=0)
    As = eye * jnp.reshape(as1, (-1, 1))
    Ad = eye * jnp.reshape(ad1, (-1, 1))
    As2 = as2.T
    Ad2 = ad2.T
    zer1 = jnp.zeros((STRIPE, W1T), jnp.float32)
    zer2 = jnp.zeros((STRIPE, W2T), jnp.float32)

    us, vs = [], []
    for t in range(T):
        tab1, edp1, mx1 = _dense1(xp[t], W1, As, Ad)
        m2d1 = jnp.zeros((16, 16), jnp.float32)
        for h in range(HEADS):
            m2d1 = m2d1.at[h].set(mx1[0, h] + mx1[0, HEADS + h])
        p0, p1 = _gat_scatter1(src, dst, tab1, edp1, m2d1, zer1)
        parts1 = jnp.stack([p0, p1])
        tab2, edp2, mx2 = _combine1_dense2(parts1, b1[None], W2, As2, Ad2)
        m2d2 = jnp.zeros((16, 16), jnp.float32).at[0].set(mx2[0, 0]
                                                          + mx2[0, 1])
        q0, q1 = _gat_scatter2(src, dst, tab2, edp2, m2d2, zer2)
        parts2 = jnp.stack([q0, q1])
        emb = _combine2(parts2, b2[None])
        u, v = _edge_gather(src, dst, emb)
        us.append(u)
        vs.append(v)
    return _gru_head(us, vs, Wih0, Whh0, bih0, bhh0, Wih1, Whh1, bih1,
                     bhh1, Wd1, bd1, Wd2, bd2)


# speedup vs baseline: 15.4875x; 1.0751x over previous
import functools

import jax
import jax.numpy as jnp
from jax import lax
from jax.experimental import pallas as pl
from jax.experimental.pallas import tpu as pltpu
from jax.experimental.pallas import tpu_sc as plsc

N = 10000
E = 160000
T = 4
D = 128
H1 = 32
HEADS = 4
GH = 64

NP = 10112
BN = 2528
W1T = 144
W2T = 48
EK = 128
NCH = E // EK
NTILES = 32
STRIPE = NP // 16


def _k1_body(x, w1, asb, adb, tab, edp, mx):
    i = pl.program_id(0)
    h = jnp.dot(x[...], w1[...], preferred_element_type=jnp.float32)
    es = jnp.dot(h, asb[...], preferred_element_type=jnp.float32)
    ed = jnp.dot(h, adb[...], preferred_element_type=jnp.float32)
    nh = es.shape[1]
    tab[...] = jnp.concatenate(
        [h, es, jnp.zeros((h.shape[0], W1T - D - nh), jnp.float32)], axis=1)
    edp[...] = jnp.concatenate(
        [ed, jnp.zeros((h.shape[0], 16 - nh), jnp.float32)], axis=1)
    bm = jnp.concatenate([jnp.max(es, axis=0), jnp.max(ed, axis=0),
                          jnp.zeros((16 - 2 * nh,), jnp.float32)])[None]

    @pl.when(i == 0)
    def _():
        mx[...] = bm

    @pl.when(i != 0)
    def _():
        mx[...] = jnp.maximum(mx[...], bm)


def _dense1(xp, W1, As, Ad):
    return pl.pallas_call(
        _k1_body,
        grid=(NP // BN,),
        in_specs=[
            pl.BlockSpec((BN, D), lambda i: (i, 0)),
            pl.BlockSpec((D, D), lambda i: (0, 0)),
            pl.BlockSpec((D, HEADS), lambda i: (0, 0)),
            pl.BlockSpec((D, HEADS), lambda i: (0, 0)),
        ],
        out_specs=[
            pl.BlockSpec((BN, W1T), lambda i: (i, 0)),
            pl.BlockSpec((BN, 16), lambda i: (i, 0)),
            pl.BlockSpec((1, 16), lambda i: (0, 0)),
        ],
        out_shape=[
            jax.ShapeDtypeStruct((NP, W1T), jnp.float32),
            jax.ShapeDtypeStruct((NP, 16), jnp.float32),
            jax.ShapeDtypeStruct((1, 16), jnp.float32),
        ],
    )(xp, W1, As, Ad)


def _k4_body(p, b1, w2, asb, adb, tab, edp, mx):
    i = pl.program_id(0)
    agg = p[0] + p[1]
    num = agg[:, :D].reshape(-1, HEADS, H1)
    den = agg[:, D:D + HEADS] + 1e-16
    o = (num / den[..., None]).reshape(-1, D) + b1[...]
    o = jnp.where(o > 0, o, jnp.exp(o) - 1.0)
    hh = jnp.dot(o, w2[...], preferred_element_type=jnp.float32)
    es = jnp.dot(hh, asb[...], preferred_element_type=jnp.float32)
    ed = jnp.dot(hh, adb[...], preferred_element_type=jnp.float32)
    tab[...] = jnp.concatenate(
        [hh, es, jnp.zeros((hh.shape[0], W2T - H1 - 1), jnp.float32)], axis=1)
    edp[...] = jnp.concatenate(
        [ed, jnp.zeros((hh.shape[0], 15), jnp.float32)], axis=1)
    bm = jnp.concatenate([jnp.max(es, axis=0), jnp.max(ed, axis=0),
                          jnp.zeros((14,), jnp.float32)])[None]

    @pl.when(i == 0)
    def _():
        mx[...] = bm

    @pl.when(i != 0)
    def _():
        mx[...] = jnp.maximum(mx[...], bm)


def _combine1_dense2(parts, b1, W2, As2, Ad2):
    return pl.pallas_call(
        _k4_body,
        grid=(NP // BN,),
        in_specs=[
            pl.BlockSpec((2, BN, W1T), lambda i: (0, i, 0)),
            pl.BlockSpec((1, D), lambda i: (0, 0)),
            pl.BlockSpec((D, H1), lambda i: (0, 0)),
            pl.BlockSpec((H1, 1), lambda i: (0, 0)),
            pl.BlockSpec((H1, 1), lambda i: (0, 0)),
        ],
        out_specs=[
            pl.BlockSpec((BN, W2T), lambda i: (i, 0)),
            pl.BlockSpec((BN, 16), lambda i: (i, 0)),
            pl.BlockSpec((1, 16), lambda i: (0, 0)),
        ],
        out_shape=[
            jax.ShapeDtypeStruct((NP, W2T), jnp.float32),
            jax.ShapeDtypeStruct((NP, 16), jnp.float32),
            jax.ShapeDtypeStruct((1, 16), jnp.float32),
        ],
    )(parts, b1, W2, As2, Ad2)


def _k6_body(p, b2, out):
    agg = p[0] + p[1]
    num = agg[:, :H1]
    den = agg[:, H1:H1 + 1] + 1e-16
    out[...] = num / den + b2[...]


def _combine2(parts, b2):
    return pl.pallas_call(
        _k6_body,
        grid=(NP // BN,),
        in_specs=[
            pl.BlockSpec((2, BN, W2T), lambda i: (0, i, 0)),
            pl.BlockSpec((1, H1), lambda i: (0, 0)),
        ],
        out_specs=pl.BlockSpec((BN, H1), lambda i: (i, 0)),
        out_shape=jax.ShapeDtypeStruct((NP, H1), jnp.float32),
    )(parts, b2)


def _make_gat_scatter(width, heads):
    ncolv = D if width == W1T else H1
    hc = ncolv // heads
    escol = ncolv
    mesh = plsc.VectorSubcoreMesh(core_axis_name="c", subcore_axis_name="s", num_cores=2, num_subcores=16)

    @functools.partial(
        pl.kernel,
        out_type=(jax.ShapeDtypeStruct((NP, width), jnp.float32),
                  jax.ShapeDtypeStruct((NP, width), jnp.float32)),
        mesh=mesh,
        compiler_params=pltpu.CompilerParams(use_tc_tiling_on_sc=False, needs_layout_passes=False),
        scratch_types=[
            pltpu.VMEM((EK,), jnp.int32),
            pltpu.VMEM((EK,), jnp.int32),
            pltpu.VMEM((EK, width), jnp.float32),
            pltpu.VMEM((EK, 16), jnp.float32),
            pltpu.VMEM((16, 16), jnp.float32),
            pltpu.VMEM_SHARED((NP, width), jnp.float32),
        ],
    )
    def gat_scatter(src_hbm, dst_hbm, tab_hbm, edp_hbm, mx_hbm, zer_hbm,
                    out0_hbm, out1_hbm, sidx, didx, rows, edr, mxv, acc):
        c = lax.axis_index("c")
        s = lax.axis_index("s")
        wid = s * 2 + c
        lane = lax.iota(jnp.int32, 16)

        pltpu.sync_copy(zer_hbm, acc.at[pl.ds(s * STRIPE, STRIPE)])
        pltpu.sync_copy(mx_hbm, mxv)
        mvec = [mxv[h] for h in range(heads)]
        plsc.subcore_barrier()

        nloop = (NCH + NTILES - 1) // NTILES

        def chunk(j, carry):
            cid = wid + NTILES * j

            @pl.when(cid < NCH)
            def _():
                base = cid * EK
                pltpu.sync_copy(src_hbm.at[pl.ds(base, EK)], sidx)
                pltpu.sync_copy(dst_hbm.at[pl.ds(base, EK)], didx)
                pltpu.sync_copy(tab_hbm.at[sidx], rows)
                pltpu.sync_copy(edp_hbm.at[didx], edr)
                eidsl = [g * 16 + lane for g in range(EK // 16)]
                for h in range(heads):
                    col_h = jnp.full((16,), escol + h, jnp.int32)
                    edcol = jnp.full((16,), h, jnp.int32)
                    exs = []
                    for g in range(EK // 16):
                        es = plsc.load_gather(rows, [eidsl[g], col_h])
                        ed = plsc.load_gather(edr, [eidsl[g], edcol])
                        e = es + ed
                        e = jnp.where(e > 0, e, 0.2 * e)
                        ex = jnp.exp(e - mvec[h])
                        plsc.store_scatter(rows, [eidsl[g], col_h], ex)
                        exs.append(ex)

                    def scale_col(cc, carry2, exs=exs):
                        ccol = jnp.full((16,), cc, jnp.int32)
                        for g in range(EK // 16):
                            v = plsc.load_gather(rows, [eidsl[g], ccol])
                            plsc.store_scatter(rows, [eidsl[g], ccol],
                                               v * exs[g])
                        return carry2

                    lax.fori_loop(h * hc, (h + 1) * hc, scale_col, 0)
                pltpu.sync_copy(rows, acc.at[didx], add=True)

            return carry

        lax.fori_loop(0, nloop, chunk, 0)
        plsc.subcore_barrier()

        @pl.when(c == 0)
        def _():
            pltpu.sync_copy(acc.at[pl.ds(s * STRIPE, STRIPE)],
                            out0_hbm.at[pl.ds(s * STRIPE, STRIPE)])

        @pl.when(c == 1)
        def _():
            pltpu.sync_copy(acc.at[pl.ds(s * STRIPE, STRIPE)],
                            out1_hbm.at[pl.ds(s * STRIPE, STRIPE)])

    return gat_scatter


_gat_scatter1 = _make_gat_scatter(W1T, HEADS)
_gat_scatter2 = _make_gat_scatter(W2T, 1)


def _make_edge_gather():
    mesh = plsc.VectorSubcoreMesh(core_axis_name="c", subcore_axis_name="s", num_cores=2, num_subcores=16)

    @functools.partial(
        pl.kernel,
        out_type=(jax.ShapeDtypeStruct((E, H1), jnp.float32),
                  jax.ShapeDtypeStruct((E, H1), jnp.float32)),
        mesh=mesh,
        compiler_params=pltpu.CompilerParams(use_tc_tiling_on_sc=False, needs_layout_passes=False),
        scratch_types=[
            pltpu.VMEM((EK,), jnp.int32),
            pltpu.VMEM((EK,), jnp.int32),
            pltpu.VMEM((EK, H1), jnp.float32),
            pltpu.VMEM((EK, H1), jnp.float32),
        ],
    )
    def edge_gather(src_hbm, dst_hbm, tab_hbm, u_hbm, v_hbm,
                    sidx, didx, urows, vrows):
        c = lax.axis_index("c")
        s = lax.axis_index("s")
        wid = s * 2 + c
        nloop = (NCH + NTILES - 1) // NTILES

        def chunk(j, carry):
            cid = wid + NTILES * j

            @pl.when(cid < NCH)
            def _():
                base = cid * EK
                pltpu.sync_copy(src_hbm.at[pl.ds(base, EK)], sidx)
                pltpu.sync_copy(dst_hbm.at[pl.ds(base, EK)], didx)
                pltpu.sync_copy(tab_hbm.at[sidx], urows)
                pltpu.sync_copy(tab_hbm.at[didx], vrows)
                pltpu.sync_copy(urows, u_hbm.at[pl.ds(base, EK)])
                pltpu.sync_copy(vrows, v_hbm.at[pl.ds(base, EK)])

            return carry

        lax.fori_loop(0, nloop, chunk, 0)

    return edge_gather


_edge_gather = _make_edge_gather()


def _gru_head_body(u0, u1, u2, u3, v0, v1, v2, v3, wihu0, wihv0, whh0,
                   bih0, bhh0, wih1, whh1, bih1, bhh1, wd1, bd1, wd2, bd2,
                   out_ref):
    blk = u0.shape[0]
    us = (u0[...], u1[...], u2[...], u3[...])
    vs = (v0[...], v1[...], v2[...], v3[...])
    h0 = jnp.zeros((blk, GH), jnp.float32)
    h1 = jnp.zeros((blk, GH), jnp.float32)
    w_ihu0 = wihu0[...]
    w_ihv0 = wihv0[...]
    w_hh0 = whh0[...]
    w_ih1 = wih1[...]
    w_hh1 = whh1[...]
    for t in range(T):
        gi = (jnp.dot(us[t], w_ihu0, preferred_element_type=jnp.float32)
              + jnp.dot(vs[t], w_ihv0, preferred_element_type=jnp.float32)
              + bih0[...])
        gh = jnp.dot(h0, w_hh0, preferred_element_type=jnp.float32) + bhh0[...]
        r = jax.nn.sigmoid(gi[:, :GH] + gh[:, :GH])
        z = jax.nn.sigmoid(gi[:, GH:2 * GH] + gh[:, GH:2 * GH])
        n = jnp.tanh(gi[:, 2 * GH:] + r * gh[:, 2 * GH:])
        h0 = (1.0 - z) * n + z * h0
        gi = jnp.dot(h0, w_ih1, preferred_element_type=jnp.float32) + bih1[...]
        gh = jnp.dot(h1, w_hh1, preferred_element_type=jnp.float32) + bhh1[...]
        r = jax.nn.sigmoid(gi[:, :GH] + gh[:, :GH])
        z = jax.nn.sigmoid(gi[:, GH:2 * GH] + gh[:, GH:2 * GH])
        n = jnp.tanh(gi[:, 2 * GH:] + r * gh[:, 2 * GH:])
        h1 = (1.0 - z) * n + z * h1
    hid = jax.nn.relu(jnp.dot(h1, wd1[...], preferred_element_type=jnp.float32)
                      + bd1[...])
    pred = jnp.dot(hid, wd2[...], preferred_element_type=jnp.float32) + bd2[...]
    out_ref[...] = pred


def _gru_head(us, vs, Wih0, Whh0, bih0, bhh0, Wih1, Whh1, bih1, bhh1,
              Wd1, bd1, Wd2, bd2):
    BLK = 400
    grid = (E // BLK,)
    eb = pl.BlockSpec((BLK, H1), lambda i: (i, 0))
    full = lambda shape: pl.BlockSpec(shape, lambda i: (0,) * len(shape))
    w_specs = [
        full((H1, 3 * GH)), full((H1, 3 * GH)), full((GH, 3 * GH)),
        full((1, 3 * GH)), full((1, 3 * GH)),
        full((GH, 3 * GH)), full((GH, 3 * GH)), full((1, 3 * GH)),
        full((1, 3 * GH)),
        full((GH, GH // 2)), full((1, GH // 2)), full((GH // 2, 1)),
        full((1, 1)),
    ]
    wih0t = Wih0.T
    return pl.pallas_call(
        _gru_head_body,
        grid=grid,
        in_specs=[eb] * 8 + w_specs,
        out_specs=pl.BlockSpec((BLK, 1), lambda i: (i, 0)),
        out_shape=jax.ShapeDtypeStruct((E, 1), jnp.float32),
    )(
        us[0], us[1], us[2], us[3], vs[0], vs[1], vs[2], vs[3],
        wih0t[:H1], wih0t[H1:], Whh0.T, bih0[None], bhh0[None],
        Wih1.T, Whh1.T, bih1[None], bhh1[None],
        Wd1.T, bd1[None], Wd2.T, bd2[None],
    )


def kernel(x_seq, common_edge_index, W1, as1, ad1, b1, W2, as2, ad2, b2,
           Wih0, Whh0, bih0, bhh0, Wih1, Whh1, bih1, bhh1, Wd1, bd1, Wd2,
           bd2):
    src = common_edge_index[0]
    dst = common_edge_index[1]
    xp = jnp.pad(x_seq, ((0, 0), (0, NP - N), (0, 0)))
    eye = jnp.repeat(jnp.eye(HEADS, dtype=jnp.float32), H1, axis=0)
    As = eye * jnp.reshape(as1, (-1, 1))
    Ad = eye * jnp.reshape(ad1, (-1, 1))
    As2 = as2.T
    Ad2 = ad2.T
    zer1 = jnp.zeros((STRIPE, W1T), jnp.float32)
    zer2 = jnp.zeros((STRIPE, W2T), jnp.float32)

    us, vs = [], []
    for t in range(T):
        tab1, edp1, mx1 = _dense1(xp[t], W1, As, Ad)
        m2d1 = jnp.zeros((16, 16), jnp.float32)
        for h in range(HEADS):
            m2d1 = m2d1.at[h].set(mx1[0, h] + mx1[0, HEADS + h])
        p0, p1 = _gat_scatter1(src, dst, tab1, edp1, m2d1, zer1)
        parts1 = jnp.stack([p0, p1])
        tab2, edp2, mx2 = _combine1_dense2(parts1, b1[None], W2, As2, Ad2)
        m2d2 = jnp.zeros((16, 16), jnp.float32).at[0].set(mx2[0, 0]
                                                          + mx2[0, 1])
        q0, q1 = _gat_scatter2(src, dst, tab2, edp2, m2d2, zer2)
        parts2 = jnp.stack([q0, q1])
        emb = _combine2(parts2, b2[None])
        u, v = _edge_gather(src, dst, emb)
        us.append(u)
        vs.append(v)
    return _gru_head(us, vs, Wih0, Whh0, bih0, bhh0, Wih1, Whh1, bih1,
                     bhh1, Wd1, bd1, Wd2, bd2)


# speedup vs baseline: 16.7151x; 1.0793x over previous
import functools

import jax
import jax.numpy as jnp
from jax import lax
from jax.experimental import pallas as pl
from jax.experimental.pallas import tpu as pltpu
from jax.experimental.pallas import tpu_sc as plsc

N = 10000
E = 160000
T = 4
D = 128
H1 = 32
HEADS = 4
GH = 64

NP = 10112
BN = 2528
W1T = 144
W2T = 48
EK = 128
NCH = E // EK
NTILES = 32
STRIPE = NP // 16


def _k1_body(x, w1, asb, adb, tab, edp, mx):
    i = pl.program_id(0)
    h = jnp.dot(x[...], w1[...], preferred_element_type=jnp.float32)
    es = jnp.dot(h, asb[...], preferred_element_type=jnp.float32)
    ed = jnp.dot(h, adb[...], preferred_element_type=jnp.float32)
    nh = es.shape[1]
    tab[...] = jnp.concatenate(
        [h, es, jnp.zeros((h.shape[0], W1T - D - nh), jnp.float32)], axis=1)
    edp[...] = jnp.concatenate(
        [ed, jnp.zeros((h.shape[0], 16 - nh), jnp.float32)], axis=1)
    bm = jnp.concatenate([jnp.max(es, axis=0), jnp.max(ed, axis=0),
                          jnp.zeros((16 - 2 * nh,), jnp.float32)])[None]

    @pl.when(i == 0)
    def _():
        mx[...] = bm

    @pl.when(i != 0)
    def _():
        mx[...] = jnp.maximum(mx[...], bm)


def _dense1(xp, W1, As, Ad):
    return pl.pallas_call(
        _k1_body,
        grid=(NP // BN,),
        in_specs=[
            pl.BlockSpec((BN, D), lambda i: (i, 0)),
            pl.BlockSpec((D, D), lambda i: (0, 0)),
            pl.BlockSpec((D, HEADS), lambda i: (0, 0)),
            pl.BlockSpec((D, HEADS), lambda i: (0, 0)),
        ],
        out_specs=[
            pl.BlockSpec((BN, W1T), lambda i: (i, 0)),
            pl.BlockSpec((BN, 16), lambda i: (i, 0)),
            pl.BlockSpec((1, 16), lambda i: (0, 0)),
        ],
        out_shape=[
            jax.ShapeDtypeStruct((NP, W1T), jnp.float32),
            jax.ShapeDtypeStruct((NP, 16), jnp.float32),
            jax.ShapeDtypeStruct((1, 16), jnp.float32),
        ],
    )(xp, W1, As, Ad)


def _k4_body(p, b1, w2, asb, adb, tab, edp, mx):
    i = pl.program_id(0)
    agg = p[0] + p[1]
    num = agg[:, :D].reshape(-1, HEADS, H1)
    den = agg[:, D:D + HEADS] + 1e-16
    o = (num / den[..., None]).reshape(-1, D) + b1[...]
    o = jnp.where(o > 0, o, jnp.exp(o) - 1.0)
    hh = jnp.dot(o, w2[...], preferred_element_type=jnp.float32)
    es = jnp.dot(hh, asb[...], preferred_element_type=jnp.float32)
    ed = jnp.dot(hh, adb[...], preferred_element_type=jnp.float32)
    tab[...] = jnp.concatenate(
        [hh, es, jnp.zeros((hh.shape[0], W2T - H1 - 1), jnp.float32)], axis=1)
    edp[...] = jnp.concatenate(
        [ed, jnp.zeros((hh.shape[0], 15), jnp.float32)], axis=1)
    bm = jnp.concatenate([jnp.max(es, axis=0), jnp.max(ed, axis=0),
                          jnp.zeros((14,), jnp.float32)])[None]

    @pl.when(i == 0)
    def _():
        mx[...] = bm

    @pl.when(i != 0)
    def _():
        mx[...] = jnp.maximum(mx[...], bm)


def _combine1_dense2(parts, b1, W2, As2, Ad2):
    return pl.pallas_call(
        _k4_body,
        grid=(NP // BN,),
        in_specs=[
            pl.BlockSpec((2, BN, W1T), lambda i: (0, i, 0)),
            pl.BlockSpec((1, D), lambda i: (0, 0)),
            pl.BlockSpec((D, H1), lambda i: (0, 0)),
            pl.BlockSpec((H1, 1), lambda i: (0, 0)),
            pl.BlockSpec((H1, 1), lambda i: (0, 0)),
        ],
        out_specs=[
            pl.BlockSpec((BN, W2T), lambda i: (i, 0)),
            pl.BlockSpec((BN, 16), lambda i: (i, 0)),
            pl.BlockSpec((1, 16), lambda i: (0, 0)),
        ],
        out_shape=[
            jax.ShapeDtypeStruct((NP, W2T), jnp.float32),
            jax.ShapeDtypeStruct((NP, 16), jnp.float32),
            jax.ShapeDtypeStruct((1, 16), jnp.float32),
        ],
    )(parts, b1, W2, As2, Ad2)


def _k6_body(p, b2, out):
    agg = p[0] + p[1]
    num = agg[:, :H1]
    den = agg[:, H1:H1 + 1] + 1e-16
    out[...] = num / den + b2[...]


def _combine2(parts, b2):
    return pl.pallas_call(
        _k6_body,
        grid=(NP // BN,),
        in_specs=[
            pl.BlockSpec((2, BN, W2T), lambda i: (0, i, 0)),
            pl.BlockSpec((1, H1), lambda i: (0, 0)),
        ],
        out_specs=pl.BlockSpec((BN, H1), lambda i: (i, 0)),
        out_shape=jax.ShapeDtypeStruct((NP, H1), jnp.float32),
    )(parts, b2)


def _make_gat_scatter(width, heads):
    ncolv = D if width == W1T else H1
    hc = ncolv // heads
    escol = ncolv
    mesh = plsc.VectorSubcoreMesh(core_axis_name="c", subcore_axis_name="s", num_cores=2, num_subcores=16)

    @functools.partial(
        pl.kernel,
        out_type=(jax.ShapeDtypeStruct((NP, width), jnp.float32),
                  jax.ShapeDtypeStruct((NP, width), jnp.float32)),
        mesh=mesh,
        compiler_params=pltpu.CompilerParams(use_tc_tiling_on_sc=False, needs_layout_passes=False),
        scratch_types=[
            pltpu.VMEM((EK,), jnp.int32),
            pltpu.VMEM((EK,), jnp.int32),
            pltpu.VMEM((EK, width), jnp.float32),
            pltpu.VMEM((EK, 16), jnp.float32),
            pltpu.VMEM((16, 16), jnp.float32),
            pltpu.VMEM_SHARED((NP, width), jnp.float32),
            pltpu.SemaphoreType.DMA,
            pltpu.SemaphoreType.DMA,
        ],
    )
    def gat_scatter(src_hbm, dst_hbm, tab_hbm, edp_hbm, mx_hbm, zer_hbm,
                    out0_hbm, out1_hbm, sidx, didx, rows, edr, mxv, acc,
                    sem1, sem2):
        c = lax.axis_index("c")
        s = lax.axis_index("s")
        wid = s * 2 + c
        lane = lax.iota(jnp.int32, 16)

        pltpu.sync_copy(zer_hbm, acc.at[pl.ds(s * STRIPE, STRIPE)])
        pltpu.sync_copy(mx_hbm, mxv)
        mvec = [mxv[h] for h in range(heads)]
        plsc.subcore_barrier()

        nloop = (NCH + NTILES - 1) // NTILES

        def chunk(j, carry):
            cid = wid + NTILES * j

            @pl.when(cid < NCH)
            def _():
                base = cid * EK
                ca = pltpu.async_copy(src_hbm.at[pl.ds(base, EK)], sidx, sem1)
                cb = pltpu.async_copy(dst_hbm.at[pl.ds(base, EK)], didx, sem2)
                ca.wait()
                cb.wait()
                cc_ = pltpu.async_copy(tab_hbm.at[sidx], rows, sem1)
                cd = pltpu.async_copy(edp_hbm.at[didx], edr, sem2)
                cc_.wait()
                cd.wait()
                eidsl = [g * 16 + lane for g in range(EK // 16)]
                for h in range(heads):
                    col_h = jnp.full((16,), escol + h, jnp.int32)
                    edcol = jnp.full((16,), h, jnp.int32)
                    exs = []
                    for g in range(EK // 16):
                        es = plsc.load_gather(rows, [eidsl[g], col_h])
                        ed = plsc.load_gather(edr, [eidsl[g], edcol])
                        e = es + ed
                        e = jnp.where(e > 0, e, 0.2 * e)
                        ex = jnp.exp(e - mvec[h])
                        plsc.store_scatter(rows, [eidsl[g], col_h], ex)
                        exs.append(ex)

                    def scale_col(cc, carry2, exs=exs):
                        ccol = jnp.full((16,), cc, jnp.int32)
                        for g in range(EK // 16):
                            v = plsc.load_gather(rows, [eidsl[g], ccol])
                            plsc.store_scatter(rows, [eidsl[g], ccol],
                                               v * exs[g])
                        return carry2

                    lax.fori_loop(h * hc, (h + 1) * hc, scale_col, 0)
                pltpu.sync_copy(rows, acc.at[didx], add=True)

            return carry

        lax.fori_loop(0, nloop, chunk, 0)
        plsc.subcore_barrier()

        @pl.when(c == 0)
        def _():
            pltpu.sync_copy(acc.at[pl.ds(s * STRIPE, STRIPE)],
                            out0_hbm.at[pl.ds(s * STRIPE, STRIPE)])

        @pl.when(c == 1)
        def _():
            pltpu.sync_copy(acc.at[pl.ds(s * STRIPE, STRIPE)],
                            out1_hbm.at[pl.ds(s * STRIPE, STRIPE)])

    return gat_scatter


_gat_scatter1 = _make_gat_scatter(W1T, HEADS)
_gat_scatter2 = _make_gat_scatter(W2T, 1)


def _make_edge_gather():
    mesh = plsc.VectorSubcoreMesh(core_axis_name="c", subcore_axis_name="s", num_cores=2, num_subcores=16)

    @functools.partial(
        pl.kernel,
        out_type=(jax.ShapeDtypeStruct((E, H1), jnp.float32),
                  jax.ShapeDtypeStruct((E, H1), jnp.float32)),
        mesh=mesh,
        compiler_params=pltpu.CompilerParams(use_tc_tiling_on_sc=False, needs_layout_passes=False),
        scratch_types=[
            pltpu.VMEM((EK,), jnp.int32),
            pltpu.VMEM((EK,), jnp.int32),
            pltpu.VMEM((EK, H1), jnp.float32),
            pltpu.VMEM((EK, H1), jnp.float32),
        ],
    )
    def edge_gather(src_hbm, dst_hbm, tab_hbm, u_hbm, v_hbm,
                    sidx, didx, urows, vrows):
        c = lax.axis_index("c")
        s = lax.axis_index("s")
        wid = s * 2 + c
        nloop = (NCH + NTILES - 1) // NTILES

        def chunk(j, carry):
            cid = wid + NTILES * j

            @pl.when(cid < NCH)
            def _():
                base = cid * EK
                pltpu.sync_copy(src_hbm.at[pl.ds(base, EK)], sidx)
                pltpu.sync_copy(dst_hbm.at[pl.ds(base, EK)], didx)
                pltpu.sync_copy(tab_hbm.at[sidx], urows)
                pltpu.sync_copy(tab_hbm.at[didx], vrows)
                pltpu.sync_copy(urows, u_hbm.at[pl.ds(base, EK)])
                pltpu.sync_copy(vrows, v_hbm.at[pl.ds(base, EK)])

            return carry

        lax.fori_loop(0, nloop, chunk, 0)

    return edge_gather


_edge_gather = _make_edge_gather()


def _gru_head_body(u0, u1, u2, u3, v0, v1, v2, v3, wihu0, wihv0, whh0,
                   bih0, bhh0, wih1, whh1, bih1, bhh1, wd1, bd1, wd2, bd2,
                   out_ref):
    blk = u0.shape[0]
    us = (u0[...], u1[...], u2[...], u3[...])
    vs = (v0[...], v1[...], v2[...], v3[...])
    h0 = jnp.zeros((blk, GH), jnp.float32)
    h1 = jnp.zeros((blk, GH), jnp.float32)
    w_ihu0 = wihu0[...]
    w_ihv0 = wihv0[...]
    w_hh0 = whh0[...]
    w_ih1 = wih1[...]
    w_hh1 = whh1[...]
    for t in range(T):
        gi = (jnp.dot(us[t], w_ihu0, preferred_element_type=jnp.float32)
              + jnp.dot(vs[t], w_ihv0, preferred_element_type=jnp.float32)
              + bih0[...])
        gh = jnp.dot(h0, w_hh0, preferred_element_type=jnp.float32) + bhh0[...]
        r = jax.nn.sigmoid(gi[:, :GH] + gh[:, :GH])
        z = jax.nn.sigmoid(gi[:, GH:2 * GH] + gh[:, GH:2 * GH])
        n = jnp.tanh(gi[:, 2 * GH:] + r * gh[:, 2 * GH:])
        h0 = (1.0 - z) * n + z * h0
        gi = jnp.dot(h0, w_ih1, preferred_element_type=jnp.float32) + bih1[...]
        gh = jnp.dot(h1, w_hh1, preferred_element_type=jnp.float32) + bhh1[...]
        r = jax.nn.sigmoid(gi[:, :GH] + gh[:, :GH])
        z = jax.nn.sigmoid(gi[:, GH:2 * GH] + gh[:, GH:2 * GH])
        n = jnp.tanh(gi[:, 2 * GH:] + r * gh[:, 2 * GH:])
        h1 = (1.0 - z) * n + z * h1
    hid = jax.nn.relu(jnp.dot(h1, wd1[...], preferred_element_type=jnp.float32)
                      + bd1[...])
    pred = jnp.dot(hid, wd2[...], preferred_element_type=jnp.float32) + bd2[...]
    out_ref[...] = pred


def _gru_head(us, vs, Wih0, Whh0, bih0, bhh0, Wih1, Whh1, bih1, bhh1,
              Wd1, bd1, Wd2, bd2):
    BLK = 400
    grid = (E // BLK,)
    eb = pl.BlockSpec((BLK, H1), lambda i: (i, 0))
    full = lambda shape: pl.BlockSpec(shape, lambda i: (0,) * len(shape))
    w_specs = [
        full((H1, 3 * GH)), full((H1, 3 * GH)), full((GH, 3 * GH)),
        full((1, 3 * GH)), full((1, 3 * GH)),
        full((GH, 3 * GH)), full((GH, 3 * GH)), full((1, 3 * GH)),
        full((1, 3 * GH)),
        full((GH, GH // 2)), full((1, GH // 2)), full((GH // 2, 1)),
        full((1, 1)),
    ]
    wih0t = Wih0.T
    return pl.pallas_call(
        _gru_head_body,
        grid=grid,
        in_specs=[eb] * 8 + w_specs,
        out_specs=pl.BlockSpec((BLK, 1), lambda i: (i, 0)),
        out_shape=jax.ShapeDtypeStruct((E, 1), jnp.float32),
    )(
        us[0], us[1], us[2], us[3], vs[0], vs[1], vs[2], vs[3],
        wih0t[:H1], wih0t[H1:], Whh0.T, bih0[None], bhh0[None],
        Wih1.T, Whh1.T, bih1[None], bhh1[None],
        Wd1.T, bd1[None], Wd2.T, bd2[None],
    )


def kernel(x_seq, common_edge_index, W1, as1, ad1, b1, W2, as2, ad2, b2,
           Wih0, Whh0, bih0, bhh0, Wih1, Whh1, bih1, bhh1, Wd1, bd1, Wd2,
           bd2):
    src = common_edge_index[0]
    dst = common_edge_index[1]
    xp = jnp.pad(x_seq, ((0, 0), (0, NP - N), (0, 0)))
    eye = jnp.repeat(jnp.eye(HEADS, dtype=jnp.float32), H1, axis=0)
    As = eye * jnp.reshape(as1, (-1, 1))
    Ad = eye * jnp.reshape(ad1, (-1, 1))
    As2 = as2.T
    Ad2 = ad2.T
    zer1 = jnp.zeros((STRIPE, W1T), jnp.float32)
    zer2 = jnp.zeros((STRIPE, W2T), jnp.float32)

    us, vs = [], []
    for t in range(T):
        tab1, edp1, mx1 = _dense1(xp[t], W1, As, Ad)
        m2d1 = jnp.zeros((16, 16), jnp.float32)
        for h in range(HEADS):
            m2d1 = m2d1.at[h].set(mx1[0, h] + mx1[0, HEADS + h])
        p0, p1 = _gat_scatter1(src, dst, tab1, edp1, m2d1, zer1)
        parts1 = jnp.stack([p0, p1])
        tab2, edp2, mx2 = _combine1_dense2(parts1, b1[None], W2, As2, Ad2)
        m2d2 = jnp.zeros((16, 16), jnp.float32).at[0].set(mx2[0, 0]
                                                          + mx2[0, 1])
        q0, q1 = _gat_scatter2(src, dst, tab2, edp2, m2d2, zer2)
        parts2 = jnp.stack([q0, q1])
        emb = _combine2(parts2, b2[None])
        u, v = _edge_gather(src, dst, emb)
        us.append(u)
        vs.append(v)
    return _gru_head(us, vs, Wih0, Whh0, bih0, bhh0, Wih1, Whh1, bih1,
                     bhh1, Wd1, bd1, Wd2, bd2)


# speedup vs baseline: 17.4399x; 1.0434x over previous
import functools

import jax
import jax.numpy as jnp
from jax import lax
from jax.experimental import pallas as pl
from jax.experimental.pallas import tpu as pltpu
from jax.experimental.pallas import tpu_sc as plsc

N = 10000
E = 160000
T = 4
D = 128
H1 = 32
HEADS = 4
GH = 64

NP = 10112
BN = 2528
W1T = 144
W2T = 48
EK = 128
NCH = E // EK
NTILES = 32
STRIPE = NP // 16


def _k1_body(x, w1, asb, adb, tab, edp, mx):
    i = pl.program_id(0)
    h = jnp.dot(x[...], w1[...], preferred_element_type=jnp.float32)
    es = jnp.dot(h, asb[...], preferred_element_type=jnp.float32)
    ed = jnp.dot(h, adb[...], preferred_element_type=jnp.float32)
    nh = es.shape[1]
    tab[...] = jnp.concatenate(
        [h, es, jnp.zeros((h.shape[0], W1T - D - nh), jnp.float32)], axis=1)
    edp[...] = jnp.concatenate(
        [ed, jnp.zeros((h.shape[0], 16 - nh), jnp.float32)], axis=1)
    bm = jnp.concatenate([jnp.max(es, axis=0), jnp.max(ed, axis=0),
                          jnp.zeros((16 - 2 * nh,), jnp.float32)])[None]

    @pl.when(i == 0)
    def _():
        mx[...] = bm

    @pl.when(i != 0)
    def _():
        mx[...] = jnp.maximum(mx[...], bm)


def _dense1(xp, W1, As, Ad):
    return pl.pallas_call(
        _k1_body,
        grid=(NP // BN,),
        in_specs=[
            pl.BlockSpec((BN, D), lambda i: (i, 0)),
            pl.BlockSpec((D, D), lambda i: (0, 0)),
            pl.BlockSpec((D, HEADS), lambda i: (0, 0)),
            pl.BlockSpec((D, HEADS), lambda i: (0, 0)),
        ],
        out_specs=[
            pl.BlockSpec((BN, W1T), lambda i: (i, 0)),
            pl.BlockSpec((BN, 16), lambda i: (i, 0)),
            pl.BlockSpec((1, 16), lambda i: (0, 0)),
        ],
        out_shape=[
            jax.ShapeDtypeStruct((NP, W1T), jnp.float32),
            jax.ShapeDtypeStruct((NP, 16), jnp.float32),
            jax.ShapeDtypeStruct((1, 16), jnp.float32),
        ],
    )(xp, W1, As, Ad)


def _k4_body(p, b1, w2, asb, adb, tab, edp, mx):
    i = pl.program_id(0)
    agg = p[0] + p[1]
    num = agg[:, :D].reshape(-1, HEADS, H1)
    den = agg[:, D:D + HEADS] + 1e-16
    o = (num / den[..., None]).reshape(-1, D) + b1[...]
    o = jnp.where(o > 0, o, jnp.exp(o) - 1.0)
    hh = jnp.dot(o, w2[...], preferred_element_type=jnp.float32)
    es = jnp.dot(hh, asb[...], preferred_element_type=jnp.float32)
    ed = jnp.dot(hh, adb[...], preferred_element_type=jnp.float32)
    tab[...] = jnp.concatenate(
        [hh, es, jnp.zeros((hh.shape[0], W2T - H1 - 1), jnp.float32)], axis=1)
    edp[...] = jnp.concatenate(
        [ed, jnp.zeros((hh.shape[0], 15), jnp.float32)], axis=1)
    bm = jnp.concatenate([jnp.max(es, axis=0), jnp.max(ed, axis=0),
                          jnp.zeros((14,), jnp.float32)])[None]

    @pl.when(i == 0)
    def _():
        mx[...] = bm

    @pl.when(i != 0)
    def _():
        mx[...] = jnp.maximum(mx[...], bm)


def _combine1_dense2(parts, b1, W2, As2, Ad2):
    return pl.pallas_call(
        _k4_body,
        grid=(NP // BN,),
        in_specs=[
            pl.BlockSpec((2, BN, W1T), lambda i: (0, i, 0)),
            pl.BlockSpec((1, D), lambda i: (0, 0)),
            pl.BlockSpec((D, H1), lambda i: (0, 0)),
            pl.BlockSpec((H1, 1), lambda i: (0, 0)),
            pl.BlockSpec((H1, 1), lambda i: (0, 0)),
        ],
        out_specs=[
            pl.BlockSpec((BN, W2T), lambda i: (i, 0)),
            pl.BlockSpec((BN, 16), lambda i: (i, 0)),
            pl.BlockSpec((1, 16), lambda i: (0, 0)),
        ],
        out_shape=[
            jax.ShapeDtypeStruct((NP, W2T), jnp.float32),
            jax.ShapeDtypeStruct((NP, 16), jnp.float32),
            jax.ShapeDtypeStruct((1, 16), jnp.float32),
        ],
    )(parts, b1, W2, As2, Ad2)


def _k6_body(p, b2, out):
    agg = p[0] + p[1]
    num = agg[:, :H1]
    den = agg[:, H1:H1 + 1] + 1e-16
    out[...] = num / den + b2[...]


def _combine2(parts, b2):
    return pl.pallas_call(
        _k6_body,
        grid=(NP // BN,),
        in_specs=[
            pl.BlockSpec((2, BN, W2T), lambda i: (0, i, 0)),
            pl.BlockSpec((1, H1), lambda i: (0, 0)),
        ],
        out_specs=pl.BlockSpec((BN, H1), lambda i: (i, 0)),
        out_shape=jax.ShapeDtypeStruct((NP, H1), jnp.float32),
    )(parts, b2)


def _make_gat_scatter(width, heads):
    ncolv = D if width == W1T else H1
    hc = ncolv // heads
    escol = ncolv
    mesh = plsc.VectorSubcoreMesh(core_axis_name="c", subcore_axis_name="s", num_cores=2, num_subcores=16)

    @functools.partial(
        pl.kernel,
        out_type=(jax.ShapeDtypeStruct((NP, width), jnp.float32),
                  jax.ShapeDtypeStruct((NP, width), jnp.float32)),
        mesh=mesh,
        compiler_params=pltpu.CompilerParams(use_tc_tiling_on_sc=False, needs_layout_passes=False),
        scratch_types=[
            pltpu.VMEM((EK,), jnp.int32),
            pltpu.VMEM((EK,), jnp.int32),
            pltpu.VMEM((EK, width), jnp.float32),
            pltpu.VMEM((EK, 16), jnp.float32),
            pltpu.VMEM((16, 16), jnp.float32),
            pltpu.VMEM_SHARED((NP, width), jnp.float32),
            pltpu.SemaphoreType.DMA,
            pltpu.SemaphoreType.DMA,
        ],
    )
    def gat_scatter(src_hbm, dst_hbm, tab_hbm, edp_hbm, mx_hbm, zer_hbm,
                    out0_hbm, out1_hbm, sidx, didx, rows, edr, mxv, acc,
                    sem1, sem2):
        c = lax.axis_index("c")
        s = lax.axis_index("s")
        wid = s * 2 + c
        lane = lax.iota(jnp.int32, 16)

        pltpu.sync_copy(zer_hbm, acc.at[pl.ds(s * STRIPE, STRIPE)])
        pltpu.sync_copy(mx_hbm, mxv)
        mvec = [mxv[h] for h in range(heads)]
        plsc.subcore_barrier()

        nloop = (NCH + NTILES - 1) // NTILES

        def chunk(j, carry):
            cid = wid + NTILES * j

            @pl.when(cid < NCH)
            def _():
                base = cid * EK
                ca = pltpu.async_copy(src_hbm.at[pl.ds(base, EK)], sidx, sem1)
                cb = pltpu.async_copy(dst_hbm.at[pl.ds(base, EK)], didx, sem2)
                ca.wait()
                cb.wait()
                cc_ = pltpu.async_copy(tab_hbm.at[sidx], rows, sem1)
                cd = pltpu.async_copy(edp_hbm.at[didx], edr, sem2)
                cc_.wait()
                cd.wait()
                eidsl = [g * 16 + lane for g in range(EK // 16)]
                for h in range(heads):
                    col_h = jnp.full((16,), escol + h, jnp.int32)
                    edcol = jnp.full((16,), h, jnp.int32)
                    exs = []
                    for g in range(EK // 16):
                        es = plsc.load_gather(rows, [eidsl[g], col_h])
                        ed = plsc.load_gather(edr, [eidsl[g], edcol])
                        e = es + ed
                        e = jnp.where(e > 0, e, 0.2 * e)
                        ex = jnp.exp(e - mvec[h])
                        plsc.store_scatter(rows, [eidsl[g], col_h], ex)
                        exs.append(ex)

                    def scale_col(cc, carry2, exs=exs):
                        ccol = jnp.full((16,), cc, jnp.int32)
                        for g in range(EK // 16):
                            v = plsc.load_gather(rows, [eidsl[g], ccol])
                            plsc.store_scatter(rows, [eidsl[g], ccol],
                                               v * exs[g])
                        return carry2

                    lax.fori_loop(h * hc, (h + 1) * hc, scale_col, 0)
                pltpu.sync_copy(rows, acc.at[didx], add=True)

            return carry

        lax.fori_loop(0, nloop, chunk, 0)
        plsc.subcore_barrier()

        @pl.when(c == 0)
        def _():
            pltpu.sync_copy(acc.at[pl.ds(s * STRIPE, STRIPE)],
                            out0_hbm.at[pl.ds(s * STRIPE, STRIPE)])

        @pl.when(c == 1)
        def _():
            pltpu.sync_copy(acc.at[pl.ds(s * STRIPE, STRIPE)],
                            out1_hbm.at[pl.ds(s * STRIPE, STRIPE)])

    return gat_scatter


_gat_scatter1 = _make_gat_scatter(W1T, HEADS)
_gat_scatter2 = _make_gat_scatter(W2T, 1)


def _make_edge_gather():
    mesh = plsc.VectorSubcoreMesh(core_axis_name="c", subcore_axis_name="s", num_cores=2, num_subcores=16)

    @functools.partial(
        pl.kernel,
        out_type=(jax.ShapeDtypeStruct((E, H1), jnp.float32),
                  jax.ShapeDtypeStruct((E, H1), jnp.float32)),
        mesh=mesh,
        compiler_params=pltpu.CompilerParams(use_tc_tiling_on_sc=False, needs_layout_passes=False),
        scratch_types=[
            pltpu.VMEM((EK,), jnp.int32),
            pltpu.VMEM((EK,), jnp.int32),
            pltpu.VMEM((EK, H1), jnp.float32),
            pltpu.VMEM((EK, H1), jnp.float32),
            pltpu.SemaphoreType.DMA,
            pltpu.SemaphoreType.DMA,
        ],
    )
    def edge_gather(src_hbm, dst_hbm, tab_hbm, u_hbm, v_hbm,
                    sidx, didx, urows, vrows, sem1, sem2):
        c = lax.axis_index("c")
        s = lax.axis_index("s")
        wid = s * 2 + c
        nloop = (NCH + NTILES - 1) // NTILES

        def chunk(j, carry):
            cid = wid + NTILES * j

            @pl.when(cid < NCH)
            def _():
                base = cid * EK
                ca = pltpu.async_copy(src_hbm.at[pl.ds(base, EK)], sidx, sem1)
                cb = pltpu.async_copy(dst_hbm.at[pl.ds(base, EK)], didx, sem2)
                ca.wait()
                cb.wait()
                cc_ = pltpu.async_copy(tab_hbm.at[sidx], urows, sem1)
                cd = pltpu.async_copy(tab_hbm.at[didx], vrows, sem2)
                cc_.wait()
                cd.wait()
                ce = pltpu.async_copy(urows, u_hbm.at[pl.ds(base, EK)], sem1)
                cf = pltpu.async_copy(vrows, v_hbm.at[pl.ds(base, EK)], sem2)
                ce.wait()
                cf.wait()

            return carry

        lax.fori_loop(0, nloop, chunk, 0)

    return edge_gather


_edge_gather = _make_edge_gather()


def _gru_head_body(u0, u1, u2, u3, v0, v1, v2, v3, wihu0, wihv0, whh0,
                   bih0, bhh0, wih1, whh1, bih1, bhh1, wd1, bd1, wd2, bd2,
                   out_ref):
    blk = u0.shape[0]
    us = (u0[...], u1[...], u2[...], u3[...])
    vs = (v0[...], v1[...], v2[...], v3[...])
    h0 = jnp.zeros((blk, GH), jnp.float32)
    h1 = jnp.zeros((blk, GH), jnp.float32)
    w_ihu0 = wihu0[...]
    w_ihv0 = wihv0[...]
    w_hh0 = whh0[...]
    w_ih1 = wih1[...]
    w_hh1 = whh1[...]
    for t in range(T):
        gi = (jnp.dot(us[t], w_ihu0, preferred_element_type=jnp.float32)
              + jnp.dot(vs[t], w_ihv0, preferred_element_type=jnp.float32)
              + bih0[...])
        gh = jnp.dot(h0, w_hh0, preferred_element_type=jnp.float32) + bhh0[...]
        r = jax.nn.sigmoid(gi[:, :GH] + gh[:, :GH])
        z = jax.nn.sigmoid(gi[:, GH:2 * GH] + gh[:, GH:2 * GH])
        n = jnp.tanh(gi[:, 2 * GH:] + r * gh[:, 2 * GH:])
        h0 = (1.0 - z) * n + z * h0
        gi = jnp.dot(h0, w_ih1, preferred_element_type=jnp.float32) + bih1[...]
        gh = jnp.dot(h1, w_hh1, preferred_element_type=jnp.float32) + bhh1[...]
        r = jax.nn.sigmoid(gi[:, :GH] + gh[:, :GH])
        z = jax.nn.sigmoid(gi[:, GH:2 * GH] + gh[:, GH:2 * GH])
        n = jnp.tanh(gi[:, 2 * GH:] + r * gh[:, 2 * GH:])
        h1 = (1.0 - z) * n + z * h1
    hid = jax.nn.relu(jnp.dot(h1, wd1[...], preferred_element_type=jnp.float32)
                      + bd1[...])
    pred = jnp.dot(hid, wd2[...], preferred_element_type=jnp.float32) + bd2[...]
    out_ref[...] = pred


def _gru_head(us, vs, Wih0, Whh0, bih0, bhh0, Wih1, Whh1, bih1, bhh1,
              Wd1, bd1, Wd2, bd2):
    BLK = 400
    grid = (E // BLK,)
    eb = pl.BlockSpec((BLK, H1), lambda i: (i, 0))
    full = lambda shape: pl.BlockSpec(shape, lambda i: (0,) * len(shape))
    w_specs = [
        full((H1, 3 * GH)), full((H1, 3 * GH)), full((GH, 3 * GH)),
        full((1, 3 * GH)), full((1, 3 * GH)),
        full((GH, 3 * GH)), full((GH, 3 * GH)), full((1, 3 * GH)),
        full((1, 3 * GH)),
        full((GH, GH // 2)), full((1, GH // 2)), full((GH // 2, 1)),
        full((1, 1)),
    ]
    wih0t = Wih0.T
    return pl.pallas_call(
        _gru_head_body,
        grid=grid,
        in_specs=[eb] * 8 + w_specs,
        out_specs=pl.BlockSpec((BLK, 1), lambda i: (i, 0)),
        out_shape=jax.ShapeDtypeStruct((E, 1), jnp.float32),
    )(
        us[0], us[1], us[2], us[3], vs[0], vs[1], vs[2], vs[3],
        wih0t[:H1], wih0t[H1:], Whh0.T, bih0[None], bhh0[None],
        Wih1.T, Whh1.T, bih1[None], bhh1[None],
        Wd1.T, bd1[None], Wd2.T, bd2[None],
    )


def kernel(x_seq, common_edge_index, W1, as1, ad1, b1, W2, as2, ad2, b2,
           Wih0, Whh0, bih0, bhh0, Wih1, Whh1, bih1, bhh1, Wd1, bd1, Wd2,
           bd2):
    src = common_edge_index[0]
    dst = common_edge_index[1]
    xp = jnp.pad(x_seq, ((0, 0), (0, NP - N), (0, 0)))
    eye = jnp.repeat(jnp.eye(HEADS, dtype=jnp.float32), H1, axis=0)
    As = eye * jnp.reshape(as1, (-1, 1))
    Ad = eye * jnp.reshape(ad1, (-1, 1))
    As2 = as2.T
    Ad2 = ad2.T
    zer1 = jnp.zeros((STRIPE, W1T), jnp.float32)
    zer2 = jnp.zeros((STRIPE, W2T), jnp.float32)

    us, vs = [], []
    for t in range(T):
        tab1, edp1, mx1 = _dense1(xp[t], W1, As, Ad)
        m2d1 = jnp.zeros((16, 16), jnp.float32)
        for h in range(HEADS):
            m2d1 = m2d1.at[h].set(mx1[0, h] + mx1[0, HEADS + h])
        p0, p1 = _gat_scatter1(src, dst, tab1, edp1, m2d1, zer1)
        parts1 = jnp.stack([p0, p1])
        tab2, edp2, mx2 = _combine1_dense2(parts1, b1[None], W2, As2, Ad2)
        m2d2 = jnp.zeros((16, 16), jnp.float32).at[0].set(mx2[0, 0]
                                                          + mx2[0, 1])
        q0, q1 = _gat_scatter2(src, dst, tab2, edp2, m2d2, zer2)
        parts2 = jnp.stack([q0, q1])
        emb = _combine2(parts2, b2[None])
        u, v = _edge_gather(src, dst, emb)
        us.append(u)
        vs.append(v)
    return _gru_head(us, vs, Wih0, Whh0, bih0, bhh0, Wih1, Whh1, bih1,
                     bhh1, Wd1, bd1, Wd2, bd2)
